# Initial kernel scaffold; baseline (speedup 1.0000x reference)
import jax
import jax.numpy as jnp
from jax.experimental import pallas as pl


def kernel(x, edge_index, W1, b1, W2, b2, W3, b3):
    raise NotImplementedError("write your pallas kernel here")


# speedup vs baseline: 21.7825x; 21.7825x over previous
import functools

import jax
import jax.numpy as jnp
from jax import lax
from jax.experimental import pallas as pl
from jax.experimental.pallas import tpu as pltpu
from jax.experimental.pallas import tpu_sc as plsc

N = 10000
D = 128
H1 = 32
H2 = 16
E = 320000

NC = 2
NS = 16
NW = NC * NS
CW = 128
KPW = 80
EPAD = NW * KPW * CW
NPAD = 10240
RPT = NPAD // NS
DEGW = 8
GRP = 8

_mesh = plsc.VectorSubcoreMesh(core_axis_name="c", subcore_axis_name="s")


@functools.partial(
    pl.kernel,
    out_type=jax.ShapeDtypeStruct((NC, NPAD, DEGW), jnp.float32),
    mesh=_mesh,
    scratch_types=[
        pltpu.VMEM((KPW, CW), jnp.int32),
        pltpu.VMEM((CW, DEGW), jnp.float32),
        pltpu.VMEM_SHARED((NPAD, DEGW), jnp.float32),
        pltpu.SemaphoreType.DMA,
    ],
    compiler_params=pltpu.CompilerParams(use_tc_tiling_on_sc=False),
)
def _deg_kernel(dst_hbm, ones_hbm, zeros_hbm, out_hbm, idx_v, ones_v, acc, sem):
    c = lax.axis_index("c")
    sid = lax.axis_index("s")
    wid = sid * NC + c
    pltpu.sync_copy(dst_hbm.at[pl.ds(wid * KPW, KPW)], idx_v)
    pltpu.sync_copy(ones_hbm, ones_v)
    pltpu.sync_copy(zeros_hbm.at[pl.ds(sid * RPT, RPT)],
                    acc.at[pl.ds(sid * RPT, RPT)])
    plsc.subcore_barrier()

    def fire(k, carry):
        pltpu.async_copy(ones_v, acc.at[idx_v.at[k]], sem, add=True)
        return carry

    lax.fori_loop(0, KPW, fire, 0)

    def drain(k, carry):
        pltpu.make_async_copy(ones_v, acc.at[idx_v.at[0]], sem).wait()
        return carry

    lax.fori_loop(0, KPW, drain, 0)
    plsc.subcore_barrier()
    pltpu.sync_copy(acc.at[pl.ds(sid * RPT, RPT)],
                    out_hbm.at[c, pl.ds(sid * RPT, RPT)])


@functools.partial(
    pl.kernel,
    out_type=jax.ShapeDtypeStruct((NC, NPAD, H1), jnp.float32),
    mesh=_mesh,
    scratch_types=[
        pltpu.VMEM((KPW, CW), jnp.int32),
        pltpu.VMEM((KPW, CW), jnp.int32),
        pltpu.VMEM((GRP, CW, H1), jnp.float32),
        pltpu.VMEM_SHARED((NPAD, H1), jnp.float32),
        pltpu.SemaphoreType.DMA((GRP,)),
        pltpu.SemaphoreType.DMA((GRP,)),
    ],
    compiler_params=pltpu.CompilerParams(use_tc_tiling_on_sc=False),
)
def _msg_kernel(tab_hbm, src_hbm, dst_hbm, zeros_hbm, out_hbm,
                src_v, dst_v, rows_v, acc, gsem, ssem):
    c = lax.axis_index("c")
    sid = lax.axis_index("s")
    wid = sid * NC + c
    pltpu.sync_copy(src_hbm.at[pl.ds(wid * KPW, KPW)], src_v)
    pltpu.sync_copy(dst_hbm.at[pl.ds(wid * KPW, KPW)], dst_v)
    pltpu.sync_copy(zeros_hbm.at[pl.ds(sid * RPT, RPT)],
                    acc.at[pl.ds(sid * RPT, RPT)])
    plsc.subcore_barrier()

    def group(g, carry):
        base = g * GRP

        @pl.when(g > 0)
        def _():
            for b in range(GRP):
                pltpu.make_async_copy(
                    rows_v.at[b], acc.at[dst_v.at[0]], ssem.at[b]).wait()

        for b in range(GRP):
            pltpu.async_copy(
                tab_hbm.at[src_v.at[base + b]], rows_v.at[b], gsem.at[b])
        for b in range(GRP):
            pltpu.make_async_copy(
                tab_hbm.at[src_v.at[0]], rows_v.at[b], gsem.at[b]).wait()
            pltpu.async_copy(
                rows_v.at[b], acc.at[dst_v.at[base + b]], ssem.at[b],
                add=True)
        return carry

    lax.fori_loop(0, KPW // GRP, group, 0)
    for b in range(GRP):
        pltpu.make_async_copy(rows_v.at[b], acc.at[dst_v.at[0]],
                              ssem.at[b]).wait()
    plsc.subcore_barrier()
    pltpu.sync_copy(acc.at[pl.ds(sid * RPT, RPT)],
                    out_hbm.at[c, pl.ds(sid * RPT, RPT)])


def _mm_body(x_ref, w_ref, o_ref):
    o_ref[...] = jnp.dot(x_ref[...], w_ref[...],
                         preferred_element_type=jnp.float32)


def _mm(x, w):
    return pl.pallas_call(
        _mm_body,
        out_shape=jax.ShapeDtypeStruct((N, w.shape[1]), jnp.float32),
    )(x, w)


def _rsqrt_deg(deg_ref):
    d = deg_ref[0, :N, 0:1] + deg_ref[1, :N, 0:1] + 1.0
    return lax.rsqrt(d)


def _scale_body(deg_ref, h_ref, o_ref):
    o_ref[...] = h_ref[...] * _rsqrt_deg(deg_ref)


def _scale(deg2, h):
    return pl.pallas_call(
        _scale_body,
        out_shape=jax.ShapeDtypeStruct((N, H1), jnp.float32),
    )(deg2, h)


def _layer1_body(agg_ref, hs_ref, deg_ref, w_ref, b_ref, o_ref):
    s = _rsqrt_deg(deg_ref)
    h1 = s * (agg_ref[0, :N, :] + agg_ref[1, :N, :] + hs_ref[...]) + b_ref[...]
    o_ref[...] = s * jnp.dot(h1, w_ref[...],
                             preferred_element_type=jnp.float32)


def _layer1(agg1, hs, deg2, wc, b1r):
    return pl.pallas_call(
        _layer1_body,
        out_shape=jax.ShapeDtypeStruct((N, H1), jnp.float32),
    )(agg1, hs, deg2, wc, b1r)


def _layer2_body(agg_ref, ts_ref, deg_ref, b_ref, mu_ref, lv_ref):
    s = _rsqrt_deg(deg_ref)
    out2 = s * (agg_ref[0, :N, :] + agg_ref[1, :N, :] + ts_ref[...]) + b_ref[...]
    mu_ref[...] = out2[:, :H2]
    lv_ref[...] = out2[:, H2:]


def _layer2(agg2, ts, deg2, bcr):
    return pl.pallas_call(
        _layer2_body,
        out_shape=(jax.ShapeDtypeStruct((N, H2), jnp.float32),
                   jax.ShapeDtypeStruct((N, H2), jnp.float32)),
    )(agg2, ts, deg2, bcr)


_BM = 400


def _dec_body(z_ref, zt_ref, o_ref):
    y = jnp.dot(z_ref[...], zt_ref[...], preferred_element_type=jnp.float32)
    o_ref[...] = 1.0 / (1.0 + jnp.exp(-y))


def _decoder(z, zt):
    return pl.pallas_call(
        _dec_body,
        grid=(N // _BM,),
        in_specs=[pl.BlockSpec((_BM, H2), lambda i: (i, 0)),
                  pl.BlockSpec((H2, N), lambda i: (0, 0))],
        out_specs=pl.BlockSpec((_BM, N), lambda i: (i, 0)),
        out_shape=jax.ShapeDtypeStruct((N, N), jnp.float32),
    )(z, zt)


def kernel(x, edge_index, W1, b1, W2, b2, W3, b3):
    src = edge_index[0]
    dst = edge_index[1]
    pad = EPAD - E
    srcp = jnp.concatenate(
        [src, jnp.zeros((pad,), jnp.int32)]).reshape(NW * KPW, CW)
    dstp = jnp.concatenate(
        [dst, jnp.full((pad,), NPAD - 1, jnp.int32)]).reshape(NW * KPW, CW)
    ones_deg = jnp.ones((CW, DEGW), jnp.float32)
    zeros_deg = jnp.zeros((NPAD, DEGW), jnp.float32)
    zeros_msg = jnp.zeros((NPAD, H1), jnp.float32)

    deg2 = _deg_kernel(dstp, ones_deg, zeros_deg)
    h = _mm(x, W1)
    hs = _scale(deg2, h)
    agg1 = _msg_kernel(hs, srcp, dstp, zeros_msg)
    wc = jnp.concatenate([W2, W3], axis=1)
    bc = jnp.concatenate([b2, b3]).reshape(1, 2 * H2)
    ts = _layer1(agg1, hs, deg2, wc, b1.reshape(1, H1))
    agg2 = _msg_kernel(ts, srcp, dstp, zeros_msg)
    mu, logvar = _layer2(agg2, ts, deg2, bc)
    adj = _decoder(mu, mu.T)
    return adj, mu, logvar


# speedup vs baseline: 32.3865x; 1.4868x over previous
import functools

import jax
import jax.numpy as jnp
from jax import lax
from jax.experimental import pallas as pl
from jax.experimental.pallas import tpu as pltpu
from jax.experimental.pallas import tpu_sc as plsc

N = 10000
D = 128
H1 = 32
H2 = 16
E = 320000

NC = 2
NS = 16
NW = NC * NS
CW = 128
KPW = 80
EPAD = NW * KPW * CW
NPAD = 10240
RPT = NPAD // NS
DEGW = 8
GRP = 8

_mesh = plsc.VectorSubcoreMesh(core_axis_name="c", subcore_axis_name="s")


@functools.partial(
    pl.kernel,
    out_type=jax.ShapeDtypeStruct((NC, NPAD, DEGW), jnp.float32),
    mesh=_mesh,
    scratch_types=[
        pltpu.VMEM((KPW, CW), jnp.int32),
        pltpu.VMEM((CW, DEGW), jnp.float32),
        pltpu.VMEM_SHARED((NPAD, DEGW), jnp.float32),
        pltpu.SemaphoreType.DMA,
    ],
    compiler_params=pltpu.CompilerParams(use_tc_tiling_on_sc=False),
)
def _deg_kernel(dst_hbm, ones_hbm, zeros_hbm, out_hbm, idx_v, ones_v, acc, sem):
    c = lax.axis_index("c")
    sid = lax.axis_index("s")
    wid = sid * NC + c
    pltpu.sync_copy(dst_hbm.at[pl.ds(wid * KPW, KPW)], idx_v)
    pltpu.sync_copy(ones_hbm, ones_v)
    pltpu.sync_copy(zeros_hbm.at[pl.ds(sid * RPT, RPT)],
                    acc.at[pl.ds(sid * RPT, RPT)])
    plsc.subcore_barrier()

    def fire(k, carry):
        pltpu.async_copy(ones_v, acc.at[idx_v.at[k]], sem, add=True)
        return carry

    lax.fori_loop(0, KPW, fire, 0)

    def drain(k, carry):
        pltpu.make_async_copy(ones_v, acc.at[idx_v.at[0]], sem).wait()
        return carry

    lax.fori_loop(0, KPW, drain, 0)
    plsc.subcore_barrier()
    pltpu.sync_copy(acc.at[pl.ds(sid * RPT, RPT)],
                    out_hbm.at[c, pl.ds(sid * RPT, RPT)])


@functools.partial(
    pl.kernel,
    out_type=jax.ShapeDtypeStruct((NC, NPAD, H1), jnp.float32),
    mesh=_mesh,
    scratch_types=[
        pltpu.VMEM((KPW, CW), jnp.int32),
        pltpu.VMEM((KPW, CW), jnp.int32),
        pltpu.VMEM((GRP, CW, H1), jnp.float32),
        pltpu.VMEM_SHARED((NPAD, H1), jnp.float32),
        pltpu.VMEM_SHARED((NPAD, H1), jnp.float32),
        pltpu.SemaphoreType.DMA((GRP,)),
        pltpu.SemaphoreType.DMA((GRP,)),
    ],
    compiler_params=pltpu.CompilerParams(use_tc_tiling_on_sc=False),
)
def _msg_kernel(tab_hbm, src_hbm, dst_hbm, zeros_hbm, out_hbm,
                src_v, dst_v, rows_v, acc, tab_sh, gsem, ssem):
    c = lax.axis_index("c")
    sid = lax.axis_index("s")
    wid = sid * NC + c
    pltpu.sync_copy(src_hbm.at[pl.ds(wid * KPW, KPW)], src_v)
    pltpu.sync_copy(dst_hbm.at[pl.ds(wid * KPW, KPW)], dst_v)
    pltpu.sync_copy(zeros_hbm.at[pl.ds(sid * RPT, RPT)],
                    acc.at[pl.ds(sid * RPT, RPT)])
    pltpu.sync_copy(tab_hbm.at[pl.ds(sid * RPT, RPT)],
                    tab_sh.at[pl.ds(sid * RPT, RPT)])
    plsc.subcore_barrier()

    def group(g, carry):
        base = g * GRP

        @pl.when(g > 0)
        def _():
            for b in range(GRP):
                pltpu.make_async_copy(
                    rows_v.at[b], acc.at[dst_v.at[0]], ssem.at[b]).wait()

        for b in range(GRP):
            pltpu.async_copy(
                tab_sh.at[src_v.at[base + b]], rows_v.at[b], gsem.at[b])
        for b in range(GRP):
            pltpu.make_async_copy(
                tab_sh.at[src_v.at[0]], rows_v.at[b], gsem.at[b]).wait()
            pltpu.async_copy(
                rows_v.at[b], acc.at[dst_v.at[base + b]], ssem.at[b],
                add=True)
        return carry

    lax.fori_loop(0, KPW // GRP, group, 0)
    for b in range(GRP):
        pltpu.make_async_copy(rows_v.at[b], acc.at[dst_v.at[0]],
                              ssem.at[b]).wait()
    plsc.subcore_barrier()
    pltpu.sync_copy(acc.at[pl.ds(sid * RPT, RPT)],
                    out_hbm.at[c, pl.ds(sid * RPT, RPT)])


def _mm_body(x_ref, w_ref, o_ref):
    o_ref[...] = jnp.dot(x_ref[...], w_ref[...],
                         preferred_element_type=jnp.float32)


def _mm(x, w):
    return pl.pallas_call(
        _mm_body,
        out_shape=jax.ShapeDtypeStruct((N, w.shape[1]), jnp.float32),
    )(x, w)


def _rsqrt_deg(deg_ref):
    d = deg_ref[0, :N, 0:1] + deg_ref[1, :N, 0:1] + 1.0
    return lax.rsqrt(d)


def _scale_body(deg_ref, h_ref, o_ref):
    o_ref[:N, :] = h_ref[...] * _rsqrt_deg(deg_ref)


def _scale(deg2, h):
    return pl.pallas_call(
        _scale_body,
        out_shape=jax.ShapeDtypeStruct((NPAD, H1), jnp.float32),
    )(deg2, h)


def _layer1_body(agg_ref, hs_ref, deg_ref, w_ref, b_ref, o_ref):
    s = _rsqrt_deg(deg_ref)
    h1 = (s * (agg_ref[0, :N, :] + agg_ref[1, :N, :] + hs_ref[:N, :])
          + b_ref[...])
    o_ref[:N, :] = s * jnp.dot(h1, w_ref[...],
                               preferred_element_type=jnp.float32)


def _layer1(agg1, hs, deg2, wc, b1r):
    return pl.pallas_call(
        _layer1_body,
        out_shape=jax.ShapeDtypeStruct((NPAD, H1), jnp.float32),
    )(agg1, hs, deg2, wc, b1r)


def _layer2_body(agg_ref, ts_ref, deg_ref, b_ref, mu_ref, lv_ref):
    s = _rsqrt_deg(deg_ref)
    out2 = (s * (agg_ref[0, :N, :] + agg_ref[1, :N, :] + ts_ref[:N, :])
            + b_ref[...])
    mu_ref[...] = out2[:, :H2]
    lv_ref[...] = out2[:, H2:]


def _layer2(agg2, ts, deg2, bcr):
    return pl.pallas_call(
        _layer2_body,
        out_shape=(jax.ShapeDtypeStruct((N, H2), jnp.float32),
                   jax.ShapeDtypeStruct((N, H2), jnp.float32)),
    )(agg2, ts, deg2, bcr)


_BM = 400


def _dec_body(z_ref, zt_ref, o_ref):
    y = jnp.dot(z_ref[...], zt_ref[...], preferred_element_type=jnp.float32)
    o_ref[...] = 1.0 / (1.0 + jnp.exp(-y))


def _decoder(z, zt):
    return pl.pallas_call(
        _dec_body,
        grid=(N // _BM,),
        in_specs=[pl.BlockSpec((_BM, H2), lambda i: (i, 0)),
                  pl.BlockSpec((H2, N), lambda i: (0, 0))],
        out_specs=pl.BlockSpec((_BM, N), lambda i: (i, 0)),
        out_shape=jax.ShapeDtypeStruct((N, N), jnp.float32),
    )(z, zt)


def kernel(x, edge_index, W1, b1, W2, b2, W3, b3):
    src = edge_index[0]
    dst = edge_index[1]
    pad = EPAD - E
    srcp = jnp.concatenate(
        [src, jnp.zeros((pad,), jnp.int32)]).reshape(NW * KPW, CW)
    dstp = jnp.concatenate(
        [dst, jnp.full((pad,), NPAD - 1, jnp.int32)]).reshape(NW * KPW, CW)
    ones_deg = jnp.ones((CW, DEGW), jnp.float32)
    zeros_deg = jnp.zeros((NPAD, DEGW), jnp.float32)
    zeros_msg = jnp.zeros((NPAD, H1), jnp.float32)

    deg2 = _deg_kernel(dstp, ones_deg, zeros_deg)
    h = _mm(x, W1)
    hs = _scale(deg2, h)
    agg1 = _msg_kernel(hs, srcp, dstp, zeros_msg)
    wc = jnp.concatenate([W2, W3], axis=1)
    bc = jnp.concatenate([b2, b3]).reshape(1, 2 * H2)
    ts = _layer1(agg1, hs, deg2, wc, b1.reshape(1, H1))
    agg2 = _msg_kernel(ts, srcp, dstp, zeros_msg)
    mu, logvar = _layer2(agg2, ts, deg2, bc)
    adj = _decoder(mu, mu.T)
    return adj, mu, logvar


# speedup vs baseline: 32.8008x; 1.0128x over previous
import functools

import jax
import jax.numpy as jnp
from jax import lax
from jax.experimental import pallas as pl
from jax.experimental.pallas import tpu as pltpu
from jax.experimental.pallas import tpu_sc as plsc

N = 10000
D = 128
H1 = 32
H2 = 16
E = 320000

NC = 2
NS = 16
NW = NC * NS
CW = 128
KPW = 80
EPAD = NW * KPW * CW
NPAD = 10240
RPT = NPAD // NS
LASTR = N - (NS - 1) * RPT
DEGW = 8
GRP = 8
KPC = NW * KPW // NS
GRP2 = 8

_mesh = plsc.VectorSubcoreMesh(core_axis_name="c", subcore_axis_name="s")


@functools.partial(
    pl.kernel,
    out_type=jax.ShapeDtypeStruct((NC, NPAD, DEGW), jnp.float32),
    mesh=_mesh,
    scratch_types=[
        pltpu.VMEM((KPW, CW), jnp.int32),
        pltpu.VMEM((CW, DEGW), jnp.float32),
        pltpu.VMEM_SHARED((NPAD, DEGW), jnp.float32),
        pltpu.SemaphoreType.DMA,
    ],
    compiler_params=pltpu.CompilerParams(use_tc_tiling_on_sc=False),
)
def _deg_kernel(dst_hbm, ones_hbm, zeros_hbm, out_hbm, idx_v, ones_v, acc, sem):
    c = lax.axis_index("c")
    sid = lax.axis_index("s")
    wid = sid * NC + c
    pltpu.sync_copy(dst_hbm.at[pl.ds(wid * KPW, KPW)], idx_v)
    pltpu.sync_copy(ones_hbm, ones_v)
    pltpu.sync_copy(zeros_hbm.at[pl.ds(sid * RPT, RPT)],
                    acc.at[pl.ds(sid * RPT, RPT)])
    plsc.subcore_barrier()

    def fire(k, carry):
        pltpu.async_copy(ones_v, acc.at[idx_v.at[k]], sem, add=True)
        return carry

    lax.fori_loop(0, KPW, fire, 0)

    def drain(k, carry):
        pltpu.make_async_copy(ones_v, acc.at[idx_v.at[0]], sem).wait()
        return carry

    lax.fori_loop(0, KPW, drain, 0)
    plsc.subcore_barrier()
    pltpu.sync_copy(acc.at[pl.ds(sid * RPT, RPT)],
                    out_hbm.at[c, pl.ds(sid * RPT, RPT)])


@functools.partial(
    pl.kernel,
    out_type=jax.ShapeDtypeStruct((NC, NPAD, H1), jnp.float32),
    mesh=_mesh,
    scratch_types=[
        pltpu.VMEM((KPW, CW), jnp.int32),
        pltpu.VMEM((KPW, CW), jnp.int32),
        pltpu.VMEM((GRP, CW, H1), jnp.float32),
        pltpu.VMEM_SHARED((NPAD, H1), jnp.float32),
        pltpu.VMEM_SHARED((NPAD, H1), jnp.float32),
        pltpu.SemaphoreType.DMA((GRP,)),
        pltpu.SemaphoreType.DMA((GRP,)),
    ],
    compiler_params=pltpu.CompilerParams(use_tc_tiling_on_sc=False),
)
def _msg_kernel(tab_hbm, src_hbm, dst_hbm, zeros_hbm, out_hbm,
                src_v, dst_v, rows_v, acc, tab_sh, gsem, ssem):
    c = lax.axis_index("c")
    sid = lax.axis_index("s")
    wid = sid * NC + c
    pltpu.sync_copy(src_hbm.at[pl.ds(wid * KPW, KPW)], src_v)
    pltpu.sync_copy(dst_hbm.at[pl.ds(wid * KPW, KPW)], dst_v)
    pltpu.sync_copy(zeros_hbm.at[pl.ds(sid * RPT, RPT)],
                    acc.at[pl.ds(sid * RPT, RPT)])
    pltpu.sync_copy(tab_hbm.at[pl.ds(sid * RPT, RPT)],
                    tab_sh.at[pl.ds(sid * RPT, RPT)])
    plsc.subcore_barrier()

    def group(g, carry):
        base = g * GRP

        @pl.when(g > 0)
        def _():
            for b in range(GRP):
                pltpu.make_async_copy(
                    rows_v.at[b], acc.at[dst_v.at[0]], ssem.at[b]).wait()

        for b in range(GRP):
            pltpu.async_copy(
                tab_sh.at[src_v.at[base + b]], rows_v.at[b], gsem.at[b])
        for b in range(GRP):
            pltpu.make_async_copy(
                tab_sh.at[src_v.at[0]], rows_v.at[b], gsem.at[b]).wait()
            pltpu.async_copy(
                rows_v.at[b], acc.at[dst_v.at[base + b]], ssem.at[b],
                add=True)
        return carry

    lax.fori_loop(0, KPW // GRP, group, 0)
    for b in range(GRP):
        pltpu.make_async_copy(rows_v.at[b], acc.at[dst_v.at[0]],
                              ssem.at[b]).wait()
    plsc.subcore_barrier()
    pltpu.sync_copy(acc.at[pl.ds(sid * RPT, RPT)],
                    out_hbm.at[c, pl.ds(sid * RPT, RPT)])


@functools.partial(
    pl.kernel,
    out_type=(jax.ShapeDtypeStruct((N, H2), jnp.float32),
              jax.ShapeDtypeStruct((N, H2), jnp.float32)),
    mesh=_mesh,
    scratch_types=[
        pltpu.VMEM((KPC, CW), jnp.int32),
        pltpu.VMEM((KPC, CW), jnp.int32),
        pltpu.VMEM((GRP2, CW, H2), jnp.float32),
        pltpu.VMEM((RPT, H2), jnp.float32),
        pltpu.VMEM((RPT, H2), jnp.float32),
        pltpu.VMEM((RPT, H2), jnp.float32),
        pltpu.VMEM((H2,), jnp.float32),
        pltpu.VMEM_SHARED((NPAD, H2), jnp.float32),
        pltpu.VMEM_SHARED((NPAD, H2), jnp.float32),
        pltpu.SemaphoreType.DMA((GRP2,)),
        pltpu.SemaphoreType.DMA((GRP2,)),
    ],
    compiler_params=pltpu.CompilerParams(use_tc_tiling_on_sc=False),
)
def _vae_head_kernel(tab_hbm, src_hbm, dst_hbm, zeros_hbm, s_hbm, bias_hbm,
                     mu_hbm, lv_hbm, src_v, dst_v, rows_v,
                     facc_v, fout_v, fs_v, fb_v, acc, tab_sh, gsem, ssem):
    c = lax.axis_index("c")
    sid = lax.axis_index("s")
    pltpu.sync_copy(src_hbm.at[pl.ds(sid * KPC, KPC)], src_v)
    pltpu.sync_copy(dst_hbm.at[pl.ds(sid * KPC, KPC)], dst_v)
    pltpu.sync_copy(zeros_hbm.at[pl.ds(sid * RPT, RPT)],
                    acc.at[pl.ds(sid * RPT, RPT)])
    pltpu.sync_copy(tab_hbm.at[c, pl.ds(sid * RPT, RPT)],
                    tab_sh.at[pl.ds(sid * RPT, RPT)])
    plsc.subcore_barrier()

    def group(g, carry):
        base = g * GRP2

        @pl.when(g > 0)
        def _():
            for b in range(GRP2):
                pltpu.make_async_copy(
                    rows_v.at[b], acc.at[dst_v.at[0]], ssem.at[b]).wait()

        for b in range(GRP2):
            pltpu.async_copy(
                tab_sh.at[src_v.at[base + b]], rows_v.at[b], gsem.at[b])
        for b in range(GRP2):
            pltpu.make_async_copy(
                tab_sh.at[src_v.at[0]], rows_v.at[b], gsem.at[b]).wait()
            pltpu.async_copy(
                rows_v.at[b], acc.at[dst_v.at[base + b]], ssem.at[b],
                add=True)
        return carry

    lax.fori_loop(0, KPC // GRP2, group, 0)
    for b in range(GRP2):
        pltpu.make_async_copy(rows_v.at[b], acc.at[dst_v.at[0]],
                              ssem.at[b]).wait()
    plsc.subcore_barrier()
    pltpu.sync_copy(acc.at[pl.ds(sid * RPT, RPT)], facc_v)
    pltpu.sync_copy(tab_sh.at[pl.ds(sid * RPT, RPT)], fout_v)
    pltpu.sync_copy(s_hbm.at[pl.ds(sid * RPT, RPT)], fs_v)
    pltpu.sync_copy(bias_hbm.at[c], fb_v)
    bias = fb_v[...]

    def frow(r, carry):
        fout_v[r, :] = fs_v[r, :] * (facc_v[r, :] + fout_v[r, :]) + bias
        return carry

    lax.fori_loop(0, RPT, frow, 0)

    @pl.when(c == 0)
    def _():
        @pl.when(sid < NS - 1)
        def _():
            pltpu.sync_copy(fout_v, mu_hbm.at[pl.ds(sid * RPT, RPT)])

        @pl.when(sid == NS - 1)
        def _():
            pltpu.sync_copy(fout_v.at[pl.ds(0, LASTR)],
                            mu_hbm.at[pl.ds((NS - 1) * RPT, LASTR)])

    @pl.when(c == 1)
    def _():
        @pl.when(sid < NS - 1)
        def _():
            pltpu.sync_copy(fout_v, lv_hbm.at[pl.ds(sid * RPT, RPT)])

        @pl.when(sid == NS - 1)
        def _():
            pltpu.sync_copy(fout_v.at[pl.ds(0, LASTR)],
                            lv_hbm.at[pl.ds((NS - 1) * RPT, LASTR)])


def _mm_body(x_ref, w_ref, o_ref):
    o_ref[...] = jnp.dot(x_ref[...], w_ref[...],
                         preferred_element_type=jnp.float32)


def _mm(x, w):
    return pl.pallas_call(
        _mm_body,
        out_shape=jax.ShapeDtypeStruct((N, w.shape[1]), jnp.float32),
    )(x, w)


def _rsqrt_deg(deg_ref):
    d = deg_ref[0, :N, 0:1] + deg_ref[1, :N, 0:1] + 1.0
    return lax.rsqrt(d)


def _scale_body(deg_ref, h_ref, o_ref):
    o_ref[:N, :] = h_ref[...] * _rsqrt_deg(deg_ref)


def _scale(deg2, h):
    return pl.pallas_call(
        _scale_body,
        out_shape=jax.ShapeDtypeStruct((NPAD, H1), jnp.float32),
    )(deg2, h)


def _layer1_body(agg_ref, hs_ref, deg_ref, w_ref, b_ref, tab_ref, s_ref):
    s = _rsqrt_deg(deg_ref)
    h1 = (s * (agg_ref[0, :N, :] + agg_ref[1, :N, :] + hs_ref[:N, :])
          + b_ref[...])
    ts = s * jnp.dot(h1, w_ref[...], preferred_element_type=jnp.float32)
    tab_ref[0, :N, :] = ts[:, :H2]
    tab_ref[1, :N, :] = ts[:, H2:]
    s_ref[:N, :] = jnp.broadcast_to(s, (N, H2))


def _layer1(agg1, hs, deg2, wc, b1r):
    return pl.pallas_call(
        _layer1_body,
        out_shape=(jax.ShapeDtypeStruct((NC, NPAD, H2), jnp.float32),
                   jax.ShapeDtypeStruct((NPAD, H2), jnp.float32)),
    )(agg1, hs, deg2, wc, b1r)


_BM = 400


def _dec_body(z_ref, zt_ref, o_ref):
    y = jnp.dot(z_ref[...], zt_ref[...], preferred_element_type=jnp.float32)
    o_ref[...] = 1.0 / (1.0 + jnp.exp(-y))


def _decoder(z, zt):
    return pl.pallas_call(
        _dec_body,
        grid=(N // _BM,),
        in_specs=[pl.BlockSpec((_BM, H2), lambda i: (i, 0)),
                  pl.BlockSpec((H2, N), lambda i: (0, 0))],
        out_specs=pl.BlockSpec((_BM, N), lambda i: (i, 0)),
        out_shape=jax.ShapeDtypeStruct((N, N), jnp.float32),
    )(z, zt)


def kernel(x, edge_index, W1, b1, W2, b2, W3, b3):
    src = edge_index[0]
    dst = edge_index[1]
    pad = EPAD - E
    srcp = jnp.concatenate(
        [src, jnp.zeros((pad,), jnp.int32)]).reshape(NW * KPW, CW)
    dstp = jnp.concatenate(
        [dst, jnp.full((pad,), NPAD - 1, jnp.int32)]).reshape(NW * KPW, CW)
    ones_deg = jnp.ones((CW, DEGW), jnp.float32)
    zeros_deg = jnp.zeros((NPAD, DEGW), jnp.float32)
    zeros_msg = jnp.zeros((NPAD, H1), jnp.float32)

    deg2 = _deg_kernel(dstp, ones_deg, zeros_deg)
    h = _mm(x, W1)
    hs = _scale(deg2, h)
    agg1 = _msg_kernel(hs, srcp, dstp, zeros_msg)
    wc = jnp.concatenate([W2, W3], axis=1)
    tab2, s16 = _layer1(agg1, hs, deg2, wc, b1.reshape(1, H1))
    zeros16 = jnp.zeros((NPAD, H2), jnp.float32)
    bias2 = jnp.stack([b2, b3])
    mu, logvar = _vae_head_kernel(tab2, srcp, dstp, zeros16, s16, bias2)
    adj = _decoder(mu, mu.T)
    return adj, mu, logvar
